# baseline (speedup 1.0000x reference)
import functools

import jax
import jax.numpy as jnp
from jax import lax
from jax.experimental import pallas as pl
from jax.experimental.pallas import tpu as pltpu
from jax.experimental.pallas import tpu_sc as plsc

N = 10000
NP = 10112
E = 320000
NW = 32
B = 96
CH = 108
CHP = CH + 2
EP = NW * CH * B
W_T = 144
RPT = NP // 16
RB = 1000
GRID = N // RB

_F = jnp.float32


def _prep_tail(xl, asv, adv, xt_ref, adst_ref, selfi_ref):
    asrc = xl @ asv
    adst = xl @ adv
    adst_ref[...] = adst
    ones = jnp.ones((xl.shape[0], 1), _F)
    zeros14 = jnp.zeros((xl.shape[0], 14), _F)
    xt_ref[...] = jnp.concatenate([xl, ones, asrc, zeros14], axis=1)
    al = asrc + adst
    al = jnp.where(al > 0, al, 0.2 * al)
    wself = jnp.exp(al)
    selfi_ref[...] = jnp.concatenate([wself * xl, wself, zeros14, ones * 0], axis=1)


def _combine(na_ref, nb_ref, b_ref):
    acc = na_ref[0] + nb_ref[0]
    den = acc[:, 128:129] + 1e-16
    return acc[:, :128] / den + b_ref[...]


def _enc_prep_body(x_ref, h_ref, ew1_ref, ew2_ref, eb_ref, w_ref, asv_ref, adv_ref,
                   z_ref, xt_ref, adst_ref, selfi_ref):
    z = x_ref[...] @ ew1_ref[...] + h_ref[...] @ ew2_ref[...] + eb_ref[...]
    z_ref[...] = z
    _prep_tail(z @ w_ref[...], asv_ref[...], adv_ref[...],
               xt_ref, adst_ref, selfi_ref)


def _mid_prep_body(na_ref, nb_ref, b_ref, w_ref, asv_ref, adv_ref,
                   xt_ref, adst_ref, selfi_ref):
    cur = jnp.maximum(_combine(na_ref, nb_ref, b_ref), 0.0)
    _prep_tail(cur @ w_ref[...], asv_ref[...], adv_ref[...],
               xt_ref, adst_ref, selfi_ref)


def _final_body(na_ref, nb_ref, b_ref, z_ref, dw1_ref, dw2_ref,
                dwb_ref, w1_ref, w1b_ref, hw_ref, hb_ref, tw_ref, tb_ref,
                hn_ref, y_ref, hsum_ref, t_ref):
    i = pl.program_id(0)
    hn = _combine(na_ref, nb_ref, b_ref)
    hn_ref[...] = hn
    o = hn @ dw1_ref[...] + z_ref[...] @ dw2_ref[...] + dwb_ref[...]
    o = jnp.maximum(o, 0.0)
    o = jnp.maximum(o @ w1_ref[...] + w1b_ref[...], 0.0)
    y_ref[...] = jax.nn.sigmoid(o @ hw_ref[...] + hb_ref[...])

    @pl.when(i == 0)
    def _():
        hsum_ref[...] = jnp.zeros_like(hsum_ref)

    hsum_ref[...] += jnp.sum(hn, axis=0, keepdims=True)

    @pl.when(i == GRID - 1)
    def _():
        t_ref[...] = jax.nn.sigmoid(
            (hsum_ref[...] * (1.0 / N)) @ tw_ref[...] + tb_ref[...])


def _row_spec(cols):
    return pl.BlockSpec((RB, cols), lambda i: (i, 0))


def _acc_spec(part):
    return pl.BlockSpec((1, RB, W_T), lambda i, p=part: (p, i, 0))


def _full_spec(r, c):
    return pl.BlockSpec((r, c), lambda i: (0, 0))


_enc_prep = pl.pallas_call(
    _enc_prep_body,
    grid=(GRID,),
    in_specs=[_row_spec(128), _row_spec(128), _full_spec(128, 128), _full_spec(128, 128),
              _full_spec(1, 128), _full_spec(128, 128), _full_spec(128, 1), _full_spec(128, 1)],
    out_specs=[_row_spec(128), _row_spec(W_T), _row_spec(1), _row_spec(W_T)],
    out_shape=[jax.ShapeDtypeStruct((N, 128), _F), jax.ShapeDtypeStruct((NP, W_T), _F),
               jax.ShapeDtypeStruct((NP, 1), _F), jax.ShapeDtypeStruct((NP, W_T), _F)],
)

_mid_prep = pl.pallas_call(
    _mid_prep_body,
    grid=(GRID,),
    in_specs=[_acc_spec(0), _acc_spec(1),
              _full_spec(1, 128), _full_spec(128, 128), _full_spec(128, 1), _full_spec(128, 1)],
    out_specs=[_row_spec(W_T), _row_spec(1), _row_spec(W_T)],
    out_shape=[jax.ShapeDtypeStruct((NP, W_T), _F),
               jax.ShapeDtypeStruct((NP, 1), _F), jax.ShapeDtypeStruct((NP, W_T), _F)],
)

_final = pl.pallas_call(
    _final_body,
    grid=(GRID,),
    in_specs=[_acc_spec(0), _acc_spec(1),
              _full_spec(1, 128), _row_spec(128), _full_spec(128, 128), _full_spec(128, 128),
              _full_spec(1, 128), _full_spec(128, 128), _full_spec(1, 128),
              _full_spec(128, 1), _full_spec(1, 1), _full_spec(128, 1), _full_spec(1, 1)],
    out_specs=[_row_spec(128), _row_spec(1), pl.BlockSpec((1, 128), lambda i: (0, 0)),
               pl.BlockSpec((1, 1), lambda i: (0, 0))],
    out_shape=[jax.ShapeDtypeStruct((N, 128), _F), jax.ShapeDtypeStruct((N, 1), _F),
               jax.ShapeDtypeStruct((1, 128), _F), jax.ShapeDtypeStruct((1, 1), _F)],
)


_sc_mesh = plsc.VectorSubcoreMesh(core_axis_name="c", subcore_axis_name="s",
                                  num_cores=2, num_subcores=16)


@functools.partial(
    pl.kernel,
    out_type=jax.ShapeDtypeStruct((2, NP, W_T), _F),
    mesh=_sc_mesh,
    compiler_params=pltpu.CompilerParams(needs_layout_passes=False,
                                         use_tc_tiling_on_sc=False),
    scratch_types=[
        pltpu.VMEM_SHARED((NP, W_T), _F),
        pltpu.VMEM((2, B), jnp.int32),
        pltpu.VMEM((2, B), jnp.int32),
        pltpu.VMEM((1, B), jnp.int32),
        pltpu.VMEM((1, B), jnp.int32),
        pltpu.VMEM((B,), _F),
        pltpu.VMEM((B,), _F),
        pltpu.VMEM((NP,), _F),
        pltpu.VMEM((B, W_T), _F),
        pltpu.VMEM((B, W_T), _F),
        pltpu.SemaphoreType.DMA,
        pltpu.SemaphoreType.DMA,
        pltpu.SemaphoreType.DMA,
        pltpu.SemaphoreType.DMA,
        pltpu.SemaphoreType.DMA,
        pltpu.SemaphoreType.DMA,
    ],
)
def _sc_edges(xt_hbm, ad_hbm, ei_hbm, si_hbm, zer_hbm, out_hbm,
              acc_sh, idx_a, idx_b, dst_a, dst_b, wbuf_a, wbuf_b, atab_d,
              rows_a, rows_b, gs_a, gs_b, is_a, is_b, ss_a, ss_b):
    c = lax.axis_index("c")
    s = lax.axis_index("s")
    wid = c * 16 + s

    pltpu.sync_copy(ad_hbm, atab_d)

    @pl.when(c == 0)
    def _():
        pltpu.sync_copy(si_hbm.at[pl.ds(s * RPT, RPT)],
                        acc_sh.at[pl.ds(s * RPT, RPT)])

    @pl.when(c != 0)
    def _():
        pltpu.sync_copy(zer_hbm.at[pl.ds(s * RPT, RPT)],
                        acc_sh.at[pl.ds(s * RPT, RPT)])

    plsc.subcore_barrier()

    iota16 = lax.iota(jnp.int32, 16)
    c129 = jnp.full((16,), 129, jnp.int32)

    def compute(idx, rows, dst):
        @plsc.parallel_loop(0, B // 16)
        def grp(v):
            base = pl.multiple_of(v * 16, 16)
            a_s = plsc.load_gather(rows, [iota16 + base, c129])
            a_d = plsc.load_gather(atab_d, [idx[1, pl.ds(base, 16)]])
            al = a_s + a_d
            al = jnp.where(al > 0, al, 0.2 * al)
            w = jnp.exp(al)
            dst[0, pl.ds(base, 16)] = idx[1, pl.ds(base, 16)]
            for e in range(16):
                we = w.at[jnp.full((16,), e, jnp.int32)].get(mode="promise_in_bounds")
                r = base + e
                for k in range(9):
                    rows[r, pl.ds(k * 16, 16)] = rows[r, pl.ds(k * 16, 16)] * we

    pltpu.sync_copy(ei_hbm.at[wid, 0], idx_a)
    pltpu.async_copy(xt_hbm.at[idx_a.at[0]], rows_a, gs_a)
    pltpu.async_copy(ei_hbm.at[wid, 1], idx_b, is_b)

    def pair(t, carry):
        j0 = 2 * t
        pltpu.make_async_copy(ei_hbm.at[wid, j0 + 1], idx_b, is_b).wait()

        @pl.when(t > 0)
        def _():
            pltpu.make_async_copy(rows_b, acc_sh.at[dst_b.at[0]], ss_b).wait()

        pltpu.async_copy(xt_hbm.at[idx_b.at[0]], rows_b, gs_b)
        pltpu.make_async_copy(xt_hbm.at[idx_a.at[0]], rows_a, gs_a).wait()
        compute(idx_a, rows_a, dst_a)
        pltpu.async_copy(rows_a, acc_sh.at[dst_a.at[0]], ss_a, add=True)
        pltpu.sync_copy(ei_hbm.at[wid, j0 + 2], idx_a)
        pltpu.make_async_copy(rows_a, acc_sh.at[dst_a.at[0]], ss_a).wait()
        pltpu.async_copy(xt_hbm.at[idx_a.at[0]], rows_a, gs_a)
        pltpu.make_async_copy(xt_hbm.at[idx_b.at[0]], rows_b, gs_b).wait()
        compute(idx_b, rows_b, dst_b)
        pltpu.async_copy(rows_b, acc_sh.at[dst_b.at[0]], ss_b, add=True)
        pltpu.async_copy(ei_hbm.at[wid, j0 + 3], idx_b, is_b)
        return carry

    lax.fori_loop(0, CH // 2, pair, 0)

    pltpu.make_async_copy(rows_b, acc_sh.at[dst_b.at[0]], ss_b).wait()
    pltpu.make_async_copy(xt_hbm.at[idx_a.at[0]], rows_a, gs_a).wait()
    pltpu.make_async_copy(ei_hbm.at[wid, CHP - 1], idx_b, is_b).wait()

    plsc.subcore_barrier()
    pltpu.sync_copy(acc_sh.at[pl.ds(s * RPT, RPT)], out_hbm.at[c, pl.ds(s * RPT, RPT)])


def kernel(x, h, edge_index, enc_w, enc_b, gat_w, gat_att_src, gat_att_dst, gat_b,
           dec_w_w, dec_w_b, dec_w1_w, dec_w1_b, dec_head_w, dec_head_b,
           term_w, term_b):
    npad = NW * CHP * B - E
    pad_idx = (N + (jnp.arange(npad, dtype=jnp.int32) % (NP - N))).astype(jnp.int32)
    nproc = NW * CH * B
    src_f = jnp.concatenate([edge_index[0], pad_idx])
    dst_f = jnp.concatenate([edge_index[1], pad_idx])
    ei_p = jnp.stack([src_f[:nproc].reshape(NW, CH, B),
                      dst_f[:nproc].reshape(NW, CH, B)], axis=2)
    extra = jnp.stack([src_f[nproc:].reshape(NW, 2, B),
                       dst_f[nproc:].reshape(NW, 2, B)], axis=2)
    ei_p = jnp.concatenate([ei_p, extra], axis=1)
    zer = jnp.zeros((NP, W_T), _F)

    eb = enc_b.reshape(1, 128)
    z, xt, adst, selfi = _enc_prep(
        x, h, enc_w[:128], enc_w[128:], eb, gat_w[0],
        gat_att_src[0].reshape(128, 1), gat_att_dst[0].reshape(128, 1))

    for i in range(3):
        out = _sc_edges(xt, adst.reshape(NP), ei_p, selfi, zer)
        wi = min(i + 1, 2)
        xt, adst, selfi = _mid_prep(
            out, out, gat_b[i].reshape(1, 128), gat_w[wi],
            gat_att_src[wi].reshape(128, 1), gat_att_dst[wi].reshape(128, 1))

    out = _sc_edges(xt, adst.reshape(NP), ei_p, selfi, zer)
    hn, y, _, t = _final(
        out, out, gat_b[2].reshape(1, 128), z,
        dec_w_w[:128], dec_w_w[128:], dec_w_b.reshape(1, 128),
        dec_w1_w, dec_w1_b.reshape(1, 128), dec_head_w, dec_head_b.reshape(1, 1),
        term_w, term_b.reshape(1, 1))
    return (y, t.reshape(1), hn)


# speedup vs baseline: 1.0541x; 1.0541x over previous
import functools

import jax
import jax.numpy as jnp
from jax import lax
from jax.experimental import pallas as pl
from jax.experimental.pallas import tpu as pltpu
from jax.experimental.pallas import tpu_sc as plsc

N = 10000
NP = 10112
E = 320000
NW = 32
B = 96
CH = 108
CHP = CH + 2
EP = NW * CH * B
W_T = 144
RPT = NP // 16
RB = 2000
GRID = N // RB

_F = jnp.float32


def _prep_tail(xl, asv, adv, xt_ref, adst_ref, selfi_ref):
    asrc = xl @ asv
    adst = xl @ adv
    adst_ref[...] = adst
    ones = jnp.ones((xl.shape[0], 1), _F)
    zeros14 = jnp.zeros((xl.shape[0], 14), _F)
    xt_ref[...] = jnp.concatenate([xl, ones, asrc, zeros14], axis=1)
    al = asrc + adst
    al = jnp.where(al > 0, al, 0.2 * al)
    wself = jnp.exp(al)
    selfi_ref[...] = jnp.concatenate([wself * xl, wself, zeros14, ones * 0], axis=1)


def _combine(na_ref, nb_ref, si_ref, b_ref):
    acc = na_ref[0] + nb_ref[0] + si_ref[...]
    den = acc[:, 128:129] + 1e-16
    return acc[:, :128] / den + b_ref[...]


def _enc_prep_body(x_ref, h_ref, ew1_ref, ew2_ref, eb_ref, w_ref, asv_ref, adv_ref,
                   z_ref, xt_ref, adst_ref, selfi_ref):
    z = x_ref[...] @ ew1_ref[...] + h_ref[...] @ ew2_ref[...] + eb_ref[...]
    z_ref[...] = z
    _prep_tail(z @ w_ref[...], asv_ref[...], adv_ref[...],
               xt_ref, adst_ref, selfi_ref)


def _mid_prep_body(na_ref, nb_ref, si_ref, b_ref, w_ref, asv_ref, adv_ref,
                   xt_ref, adst_ref, selfi_ref):
    cur = jnp.maximum(_combine(na_ref, nb_ref, si_ref, b_ref), 0.0)
    _prep_tail(cur @ w_ref[...], asv_ref[...], adv_ref[...],
               xt_ref, adst_ref, selfi_ref)


def _final_body(na_ref, nb_ref, si_ref, b_ref, z_ref, dw1_ref, dw2_ref,
                dwb_ref, w1_ref, w1b_ref, hw_ref, hb_ref, tw_ref, tb_ref,
                hn_ref, y_ref, hsum_ref, t_ref):
    i = pl.program_id(0)
    hn = _combine(na_ref, nb_ref, si_ref, b_ref)
    hn_ref[...] = hn
    o = hn @ dw1_ref[...] + z_ref[...] @ dw2_ref[...] + dwb_ref[...]
    o = jnp.maximum(o, 0.0)
    o = jnp.maximum(o @ w1_ref[...] + w1b_ref[...], 0.0)
    y_ref[...] = jax.nn.sigmoid(o @ hw_ref[...] + hb_ref[...])

    @pl.when(i == 0)
    def _():
        hsum_ref[...] = jnp.zeros_like(hsum_ref)

    hsum_ref[...] += jnp.sum(hn, axis=0, keepdims=True)

    @pl.when(i == GRID - 1)
    def _():
        t_ref[...] = jax.nn.sigmoid(
            (hsum_ref[...] * (1.0 / N)) @ tw_ref[...] + tb_ref[...])


def _row_spec(cols):
    return pl.BlockSpec((RB, cols), lambda i: (i, 0))


def _acc_spec(part):
    return pl.BlockSpec((1, RB, W_T), lambda i, p=part: (p, i, 0))


def _full_spec(r, c):
    return pl.BlockSpec((r, c), lambda i: (0, 0))


_enc_prep = pl.pallas_call(
    _enc_prep_body,
    grid=(GRID,),
    in_specs=[_row_spec(128), _row_spec(128), _full_spec(128, 128), _full_spec(128, 128),
              _full_spec(1, 128), _full_spec(128, 128), _full_spec(128, 1), _full_spec(128, 1)],
    out_specs=[_row_spec(128), _row_spec(W_T), _row_spec(1), _row_spec(W_T)],
    out_shape=[jax.ShapeDtypeStruct((N, 128), _F), jax.ShapeDtypeStruct((NP, W_T), _F),
               jax.ShapeDtypeStruct((NP, 1), _F), jax.ShapeDtypeStruct((N, W_T), _F)],
)

_mid_prep = pl.pallas_call(
    _mid_prep_body,
    grid=(GRID,),
    in_specs=[_acc_spec(0), _acc_spec(1), _row_spec(W_T),
              _full_spec(1, 128), _full_spec(128, 128), _full_spec(128, 1), _full_spec(128, 1)],
    out_specs=[_row_spec(W_T), _row_spec(1), _row_spec(W_T)],
    out_shape=[jax.ShapeDtypeStruct((NP, W_T), _F),
               jax.ShapeDtypeStruct((NP, 1), _F), jax.ShapeDtypeStruct((N, W_T), _F)],
)

_final = pl.pallas_call(
    _final_body,
    grid=(GRID,),
    in_specs=[_acc_spec(0), _acc_spec(1), _row_spec(W_T),
              _full_spec(1, 128), _row_spec(128), _full_spec(128, 128), _full_spec(128, 128),
              _full_spec(1, 128), _full_spec(128, 128), _full_spec(1, 128),
              _full_spec(128, 1), _full_spec(1, 1), _full_spec(128, 1), _full_spec(1, 1)],
    out_specs=[_row_spec(128), _row_spec(1), pl.BlockSpec((1, 128), lambda i: (0, 0)),
               pl.BlockSpec((1, 1), lambda i: (0, 0))],
    out_shape=[jax.ShapeDtypeStruct((N, 128), _F), jax.ShapeDtypeStruct((N, 1), _F),
               jax.ShapeDtypeStruct((1, 128), _F), jax.ShapeDtypeStruct((1, 1), _F)],
)


_sc_mesh = plsc.VectorSubcoreMesh(core_axis_name="c", subcore_axis_name="s",
                                  num_cores=2, num_subcores=16)


@functools.partial(
    pl.kernel,
    out_type=jax.ShapeDtypeStruct((2, NP, W_T), _F),
    mesh=_sc_mesh,
    compiler_params=pltpu.CompilerParams(needs_layout_passes=False,
                                         use_tc_tiling_on_sc=False),
    scratch_types=[
        pltpu.VMEM_SHARED((NP, W_T), _F),
        pltpu.VMEM((2, B), jnp.int32),
        pltpu.VMEM((2, B), jnp.int32),
        pltpu.VMEM((1, B), jnp.int32),
        pltpu.VMEM((1, B), jnp.int32),
        pltpu.VMEM((B,), _F),
        pltpu.VMEM((B,), _F),
        pltpu.VMEM((NP,), _F),
        pltpu.VMEM((B, W_T), _F),
        pltpu.VMEM((B, W_T), _F),
        pltpu.SemaphoreType.DMA,
        pltpu.SemaphoreType.DMA,
        pltpu.SemaphoreType.DMA,
        pltpu.SemaphoreType.DMA,
        pltpu.SemaphoreType.DMA,
        pltpu.SemaphoreType.DMA,
    ],
)
def _sc_edges(xt_hbm, ad_hbm, ei_hbm, zer_hbm, out_hbm,
              acc_sh, idx_a, idx_b, dst_a, dst_b, wbuf_a, wbuf_b, atab_d,
              rows_a, rows_b, gs_a, gs_b, is_a, is_b, ss_a, ss_b):
    c = lax.axis_index("c")
    s = lax.axis_index("s")
    wid = c * 16 + s

    pltpu.sync_copy(ad_hbm, atab_d)
    pltpu.sync_copy(zer_hbm.at[pl.ds(s * RPT, RPT)],
                    acc_sh.at[pl.ds(s * RPT, RPT)])

    plsc.subcore_barrier()

    iota16 = lax.iota(jnp.int32, 16)
    c129 = jnp.full((16,), 129, jnp.int32)

    def compute(idx, rows, dst):
        @plsc.parallel_loop(0, B // 16)
        def grp(v):
            base = pl.multiple_of(v * 16, 16)
            a_s = plsc.load_gather(rows, [iota16 + base, c129])
            a_d = plsc.load_gather(atab_d, [idx[1, pl.ds(base, 16)]])
            al = a_s + a_d
            al = jnp.where(al > 0, al, 0.2 * al)
            w = jnp.exp(al)
            dst[0, pl.ds(base, 16)] = idx[1, pl.ds(base, 16)]
            for e in range(16):
                we = w.at[jnp.full((16,), e, jnp.int32)].get(mode="promise_in_bounds")
                r = base + e
                for k in range(9):
                    rows[r, pl.ds(k * 16, 16)] = rows[r, pl.ds(k * 16, 16)] * we

    pltpu.sync_copy(ei_hbm.at[wid, 0], idx_a)
    pltpu.async_copy(xt_hbm.at[idx_a.at[0]], rows_a, gs_a)
    pltpu.async_copy(ei_hbm.at[wid, 1], idx_b, is_b)

    def pair(t, carry):
        j0 = 2 * t
        pltpu.make_async_copy(ei_hbm.at[wid, j0 + 1], idx_b, is_b).wait()

        @pl.when(t > 0)
        def _():
            pltpu.make_async_copy(rows_b, acc_sh.at[dst_b.at[0]], ss_b).wait()

        pltpu.async_copy(xt_hbm.at[idx_b.at[0]], rows_b, gs_b)
        pltpu.make_async_copy(xt_hbm.at[idx_a.at[0]], rows_a, gs_a).wait()
        compute(idx_a, rows_a, dst_a)
        pltpu.async_copy(rows_a, acc_sh.at[dst_a.at[0]], ss_a, add=True)
        pltpu.sync_copy(ei_hbm.at[wid, j0 + 2], idx_a)
        pltpu.make_async_copy(rows_a, acc_sh.at[dst_a.at[0]], ss_a).wait()
        pltpu.async_copy(xt_hbm.at[idx_a.at[0]], rows_a, gs_a)
        pltpu.make_async_copy(xt_hbm.at[idx_b.at[0]], rows_b, gs_b).wait()
        compute(idx_b, rows_b, dst_b)
        pltpu.async_copy(rows_b, acc_sh.at[dst_b.at[0]], ss_b, add=True)
        pltpu.async_copy(ei_hbm.at[wid, j0 + 3], idx_b, is_b)
        return carry

    lax.fori_loop(0, CH // 2, pair, 0)

    pltpu.make_async_copy(rows_b, acc_sh.at[dst_b.at[0]], ss_b).wait()
    pltpu.make_async_copy(xt_hbm.at[idx_a.at[0]], rows_a, gs_a).wait()
    pltpu.make_async_copy(ei_hbm.at[wid, CHP - 1], idx_b, is_b).wait()

    plsc.subcore_barrier()
    pltpu.sync_copy(acc_sh.at[pl.ds(s * RPT, RPT)], out_hbm.at[c, pl.ds(s * RPT, RPT)])


def kernel(x, h, edge_index, enc_w, enc_b, gat_w, gat_att_src, gat_att_dst, gat_b,
           dec_w_w, dec_w_b, dec_w1_w, dec_w1_b, dec_head_w, dec_head_b,
           term_w, term_b):
    npad = NW * CHP * B - E
    pad_idx = (N + (jnp.arange(npad, dtype=jnp.int32) % (NP - N))).astype(jnp.int32)
    nproc = NW * CH * B
    src_f = jnp.concatenate([edge_index[0], pad_idx])
    dst_f = jnp.concatenate([edge_index[1], pad_idx])
    ei_p = jnp.stack([src_f[:nproc].reshape(NW, CH, B),
                      dst_f[:nproc].reshape(NW, CH, B)], axis=2)
    extra = jnp.stack([src_f[nproc:].reshape(NW, 2, B),
                       dst_f[nproc:].reshape(NW, 2, B)], axis=2)
    ei_p = jnp.concatenate([ei_p, extra], axis=1)
    zer = jnp.zeros((NP, W_T), _F)

    eb = enc_b.reshape(1, 128)
    z, xt, adst, selfi = _enc_prep(
        x, h, enc_w[:128], enc_w[128:], eb, gat_w[0],
        gat_att_src[0].reshape(128, 1), gat_att_dst[0].reshape(128, 1))

    for i in range(3):
        out = _sc_edges(xt, adst.reshape(NP), ei_p, zer)
        wi = min(i + 1, 2)
        xt, adst, selfi = _mid_prep(
            out, out, selfi, gat_b[i].reshape(1, 128), gat_w[wi],
            gat_att_src[wi].reshape(128, 1), gat_att_dst[wi].reshape(128, 1))

    out = _sc_edges(xt, adst.reshape(NP), ei_p, zer)
    hn, y, _, t = _final(
        out, out, selfi, gat_b[2].reshape(1, 128), z,
        dec_w_w[:128], dec_w_w[128:], dec_w_b.reshape(1, 128),
        dec_w1_w, dec_w1_b.reshape(1, 128), dec_head_w, dec_head_b.reshape(1, 1),
        term_w, term_b.reshape(1, 1))
    return (y, t.reshape(1), hn)
